# baseline (speedup 1.0000x reference)
import functools

import jax
import jax.numpy as jnp
from jax import lax
from jax.experimental import pallas as pl
from jax.experimental.pallas import tpu as pltpu
from jax.experimental.pallas import tpu_sc as plsc

_NUM_TYPES = 32
_EMB = 16
_E = 1600000
_F = 4
_NC, _NS = 2, 16
_NW = _NC * _NS
_NT_TOTAL = _E // 128
_NT = 20
_NCHUNK = _NT_TOTAL // _NT
_PAD = 16

_mesh = plsc.VectorSubcoreMesh(core_axis_name="c", subcore_axis_name="s")


@functools.partial(
    pl.kernel,
    out_type=[
        jax.ShapeDtypeStruct((8, _NT_TOTAL, 8, 128), jnp.float32),
        jax.ShapeDtypeStruct((8, _NT_TOTAL, 8, 128), jnp.float32),
    ],
    mesh=_mesh,
    scratch_types=[
        pltpu.VMEM((_NUM_TYPES, _EMB), jnp.float32),
        pltpu.VMEM((_NUM_TYPES, _PAD), jnp.float32),
        pltpu.VMEM((_NT, _F, 128), jnp.int32),
        pltpu.VMEM((2, _NT, 8, 128), jnp.float32),
        pltpu.SemaphoreType.DMA,
        pltpu.SemaphoreType.DMA,
    ],
    compiler_params=pltpu.CompilerParams(use_tc_tiling_on_sc=False,
                                         needs_layout_passes=False),
)
def _encode(pe_idx, ple_idx, table_hbm, pe_out, ple_out,
            table_st, table_v, idx_v, out_v, sem0, sem1):
    cid = lax.axis_index("c")
    sid = lax.axis_index("s")
    wid = sid * _NC + cid

    pltpu.sync_copy(table_hbm, table_st)
    for r in range(_NUM_TYPES):
        table_v[r, pl.ds(0, _EMB)] = table_st[r, pl.ds(0, _EMB)]

    def _do(idx_hbm, out_hbm):
        def _chunk(n, carry):
            c = wid + n * _NW

            @pl.when(c < _NCHUNK)
            def _():
                t0 = pl.multiple_of(c * _NT, _NT)
                pltpu.sync_copy(idx_hbm.at[pl.ds(t0, _NT)], idx_v)
                sems = (sem0, sem1)

                def _hh(hh, carry2):
                    for p in range(2):
                        h = 2 * hh + p

                        @pl.when(hh >= 1)
                        def _wait():
                            pltpu.make_async_copy(
                                out_v.at[p],
                                out_hbm.at[h - 2, pl.ds(t0, _NT)],
                                sems[p]).wait()

                        @plsc.parallel_loop(0, _NT, step=1, unroll=4)
                        def _tile(tt):
                            for jv in range(8):
                                iv = idx_v[tt, hh, pl.ds(jv * 16, 16)]
                                vals = []
                                for l in range(8):
                                    dv = jnp.full((16,), 8 * p + l, jnp.int32)
                                    vals.append(
                                        plsc.load_gather(table_v, [iv, dv]))
                                for l in range(8):
                                    out_v[p, tt, l, pl.ds(jv * 16, 16)] = (
                                        vals[l])

                        pltpu.make_async_copy(
                            out_v.at[p],
                            out_hbm.at[h, pl.ds(t0, _NT)],
                            sems[p]).start()
                    return carry2

                lax.fori_loop(0, 4, _hh, 0)
                for p in range(2):
                    pltpu.make_async_copy(
                        out_v.at[p],
                        out_hbm.at[6 + p, pl.ds(t0, _NT)],
                        sems[p]).wait()

            return carry

        lax.fori_loop(0, (_NCHUNK + _NW - 1) // _NW, _chunk, 0)

    _do(pe_idx, pe_out)
    _do(ple_idx, ple_out)


def kernel(parent_edge_features, parent_light_edge_features, table):
    def _view_idx(idx):
        return lax.transpose(
            lax.reshape(idx, (_F, _NT_TOTAL, 128), dimensions=(1, 0)),
            (1, 0, 2))

    pe4, ple4 = _encode(_view_idx(parent_edge_features),
                        _view_idx(parent_light_edge_features),
                        table)

    def _view_out(o4):
        return lax.reshape(lax.transpose(o4, (1, 3, 0, 2)),
                           (_E, _F * _EMB))

    return (_view_out(pe4), _view_out(ple4))


# speedup vs baseline: 1.3630x; 1.3630x over previous
import functools

import jax
import jax.numpy as jnp
from jax import lax
from jax.experimental import pallas as pl
from jax.experimental.pallas import tpu as pltpu
from jax.experimental.pallas import tpu_sc as plsc

_NUM_TYPES = 32
_EMB = 16
_E = 1600000
_F = 4
_NC, _NS = 2, 16
_NW = _NC * _NS
_NT_TOTAL = _E // 128
_NT = 20
_NCHUNK = _NT_TOTAL // _NT
_PAD = 17

_mesh = plsc.VectorSubcoreMesh(core_axis_name="c", subcore_axis_name="s")


@functools.partial(
    pl.kernel,
    out_type=[
        jax.ShapeDtypeStruct((8, _NT_TOTAL, 8, 128), jnp.float32),
        jax.ShapeDtypeStruct((8, _NT_TOTAL, 8, 128), jnp.float32),
    ],
    mesh=_mesh,
    scratch_types=[
        pltpu.VMEM((_NUM_TYPES, _EMB), jnp.float32),
        pltpu.VMEM((_NUM_TYPES, _PAD), jnp.float32),
        pltpu.VMEM((_NT, _F, 128), jnp.int32),
        pltpu.VMEM((2, _NT, 8, 128), jnp.float32),
        pltpu.SemaphoreType.DMA,
        pltpu.SemaphoreType.DMA,
    ],
    compiler_params=pltpu.CompilerParams(use_tc_tiling_on_sc=False,
                                         needs_layout_passes=False),
)
def _encode(pe_idx, ple_idx, table_hbm, pe_out, ple_out,
            table_st, table_v, idx_v, out_v, sem0, sem1):
    cid = lax.axis_index("c")
    sid = lax.axis_index("s")
    wid = sid * _NC + cid

    pltpu.sync_copy(table_hbm, table_st)
    for r in range(_NUM_TYPES):
        table_v[r, pl.ds(0, _EMB)] = table_st[r, pl.ds(0, _EMB)]

    def _do(idx_hbm, out_hbm):
        def _chunk(n, carry):
            c = wid + n * _NW

            @pl.when(c < _NCHUNK)
            def _():
                t0 = pl.multiple_of(c * _NT, _NT)
                pltpu.sync_copy(idx_hbm.at[pl.ds(t0, _NT)], idx_v)
                sems = (sem0, sem1)

                def _hh(hh, carry2):
                    for p in range(2):
                        h = 2 * hh + p

                        @pl.when(hh >= 1)
                        def _wait():
                            pltpu.make_async_copy(
                                out_v.at[p],
                                out_hbm.at[h - 2, pl.ds(t0, _NT)],
                                sems[p]).wait()

                        @plsc.parallel_loop(0, _NT, step=1, unroll=8)
                        def _tile(tt):
                            for jv in range(8):
                                iv = idx_v[tt, hh, pl.ds(jv * 16, 16)]
                                vals = []
                                for l in range(8):
                                    dv = jnp.full((16,), 8 * p + l, jnp.int32)
                                    vals.append(
                                        plsc.load_gather(table_v, [iv, dv]))
                                for l in range(8):
                                    out_v[p, tt, l, pl.ds(jv * 16, 16)] = (
                                        vals[l])

                        pltpu.make_async_copy(
                            out_v.at[p],
                            out_hbm.at[h, pl.ds(t0, _NT)],
                            sems[p]).start()
                    return carry2

                lax.fori_loop(0, 4, _hh, 0)
                for p in range(2):
                    pltpu.make_async_copy(
                        out_v.at[p],
                        out_hbm.at[6 + p, pl.ds(t0, _NT)],
                        sems[p]).wait()

            return carry

        lax.fori_loop(0, (_NCHUNK + _NW - 1) // _NW, _chunk, 0)

    _do(pe_idx, pe_out)
    _do(ple_idx, ple_out)


def kernel(parent_edge_features, parent_light_edge_features, table):
    def _view_idx(idx):
        return lax.transpose(
            lax.reshape(idx, (_F, _NT_TOTAL, 128), dimensions=(1, 0)),
            (1, 0, 2))

    pe4, ple4 = _encode(_view_idx(parent_edge_features),
                        _view_idx(parent_light_edge_features),
                        table)

    def _view_out(o4):
        return lax.reshape(lax.transpose(o4, (1, 3, 0, 2)),
                           (_E, _F * _EMB))

    return (_view_out(pe4), _view_out(ple4))


# speedup vs baseline: 2.0453x; 1.5005x over previous
import functools

import jax
import jax.numpy as jnp
from jax import lax
from jax.experimental import pallas as pl
from jax.experimental.pallas import tpu as pltpu
from jax.experimental.pallas import tpu_sc as plsc

_NUM_TYPES = 32
_EMB = 16
_E = 1600000
_F = 4
_NC, _NS = 2, 16
_NW = _NC * _NS
_NT_TOTAL = _E // 128
_NT = 20
_NCHUNK = _NT_TOTAL // _NT
_PAD = 17

_mesh = plsc.VectorSubcoreMesh(core_axis_name="c", subcore_axis_name="s")


@functools.partial(
    pl.kernel,
    out_type=[
        jax.ShapeDtypeStruct((8, _NT_TOTAL, 8, 128), jnp.float32),
        jax.ShapeDtypeStruct((8, _NT_TOTAL, 8, 128), jnp.float32),
    ],
    mesh=_mesh,
    scratch_types=[
        pltpu.VMEM((_NUM_TYPES, _EMB), jnp.float32),
        pltpu.VMEM((_NUM_TYPES, _PAD), jnp.float32),
        pltpu.VMEM((2, _NT, _F, 128), jnp.int32),
        pltpu.VMEM((2, _NT, 8, 128), jnp.float32),
        pltpu.SemaphoreType.DMA,
        pltpu.SemaphoreType.DMA,
        pltpu.SemaphoreType.DMA,
    ],
    compiler_params=pltpu.CompilerParams(use_tc_tiling_on_sc=False,
                                         needs_layout_passes=False),
)
def _encode(pe_idx, ple_idx, table_hbm, pe_out, ple_out,
            table_st, table_v, idx_v, out_v, sem0, sem1, semi):
    cid = lax.axis_index("c")
    sid = lax.axis_index("s")
    wid = sid * _NC + cid

    pltpu.sync_copy(table_hbm, table_st)
    for r in range(_NUM_TYPES):
        table_v[r, pl.ds(0, _EMB)] = table_st[r, pl.ds(0, _EMB)]

    def _do(idx_hbm, out_hbm):
        pltpu.make_async_copy(
            idx_hbm.at[pl.ds(pl.multiple_of(wid * _NT, _NT), _NT)],
            idx_v.at[0], semi).start()

        def _chunk(n, carry):
            c = wid + n * _NW
            q = n % 2

            @pl.when(c < _NCHUNK)
            def _():
                t0 = pl.multiple_of(c * _NT, _NT)
                pltpu.make_async_copy(
                    idx_hbm.at[pl.ds(t0, _NT)], idx_v.at[q], semi).wait()

                @pl.when(c + _NW < _NCHUNK)
                def _prefetch():
                    t1 = pl.multiple_of((c + _NW) * _NT, _NT)
                    pltpu.make_async_copy(
                        idx_hbm.at[pl.ds(t1, _NT)], idx_v.at[1 - q],
                        semi).start()

                sems = (sem0, sem1)

                def _hh(hh, carry2):
                    for p in range(2):
                        h = 2 * hh + p

                        @pl.when(hh >= 1)
                        def _wait():
                            pltpu.make_async_copy(
                                out_v.at[p],
                                out_hbm.at[h - 2, pl.ds(t0, _NT)],
                                sems[p]).wait()

                        @plsc.parallel_loop(0, _NT, step=1, unroll=4)
                        def _tile(tt):
                            for jv in range(8):
                                iv = idx_v[q, tt, hh, pl.ds(jv * 16, 16)]
                                vals = []
                                for l in range(8):
                                    dv = jnp.full((16,), 8 * p + l, jnp.int32)
                                    vals.append(
                                        plsc.load_gather(table_v, [iv, dv]))
                                for l in range(8):
                                    out_v[p, tt, l, pl.ds(jv * 16, 16)] = (
                                        vals[l])

                        pltpu.make_async_copy(
                            out_v.at[p],
                            out_hbm.at[h, pl.ds(t0, _NT)],
                            sems[p]).start()
                    return carry2

                lax.fori_loop(0, 4, _hh, 0)
                for p in range(2):
                    pltpu.make_async_copy(
                        out_v.at[p],
                        out_hbm.at[6 + p, pl.ds(t0, _NT)],
                        sems[p]).wait()

            return carry

        lax.fori_loop(0, (_NCHUNK + _NW - 1) // _NW, _chunk, 0)

    _do(pe_idx, pe_out)
    _do(ple_idx, ple_out)


def kernel(parent_edge_features, parent_light_edge_features, table):
    def _view_idx(idx):
        return lax.transpose(
            lax.reshape(idx, (_F, _NT_TOTAL, 128), dimensions=(1, 0)),
            (1, 0, 2))

    pe4, ple4 = _encode(_view_idx(parent_edge_features),
                        _view_idx(parent_light_edge_features),
                        table)

    def _view_out(o4):
        return lax.reshape(lax.transpose(o4, (1, 3, 0, 2)),
                           (_E, _F * _EMB))

    return (_view_out(pe4), _view_out(ple4))
